# Initial kernel scaffold; baseline (speedup 1.0000x reference)
import jax
import jax.numpy as jnp
from jax.experimental import pallas as pl


def kernel(x, edge_index, weight, bias):
    raise NotImplementedError("write your pallas kernel here")


# speedup vs baseline: 8.8816x; 8.8816x over previous
import functools

import jax
import jax.numpy as jnp
from jax import lax
from jax.experimental import pallas as pl
from jax.experimental.pallas import tpu as pltpu
from jax.experimental.pallas import tpu_sc as plsc

NN = 10000
EE = 160000
DIN = 256
DOUT = 256
DH = 128
NC = 2
NS = 16
NP = 10240
CPT = NP // NS
EPT = EE // (NC * NS)
EPS = EE // NS
CH = 80
RPT = NP // NS
DRC = 128
RB = 2000
NRB = NN // RB


def _sc_mesh():
    return plsc.VectorSubcoreMesh(
        core_axis_name="c", subcore_axis_name="s",
        num_cores=NC, num_subcores=NS)


def _deg_body(src_hbm, pdeg_hbm, src_v, deg_v, stage_sh, red_v, res_v):
    c = lax.axis_index("c")
    s = lax.axis_index("s")
    wid = c * NS + s
    zeros16 = jnp.zeros((16,), jnp.float32)
    ones16 = jnp.ones((16,), jnp.float32)
    lane = lax.iota(jnp.int32, 16)

    def _zero(i, carry):
        deg_v[pl.ds(i * 16, 16)] = zeros16
        return carry
    lax.fori_loop(0, NP // 16, _zero, 0)

    pltpu.sync_copy(src_hbm.at[pl.ds(wid * EPT, EPT)],
                    src_v.at[pl.ds(0, EPT)])

    def _scat(i, carry):
        base = i * 16
        idx = src_v[pl.ds(base, 16)]
        idx = jnp.minimum(jnp.maximum(idx, 0), NP - 1)
        m = lane < (EPT - base)
        plsc.addupdate_scatter(deg_v, [idx], ones16, mask=m)
        return carry
    lax.fori_loop(0, (EPT + 15) // 16, _scat, 0)

    pltpu.sync_copy(deg_v, stage_sh.at[s])
    plsc.subcore_barrier()
    col0 = s * CPT
    pltpu.sync_copy(stage_sh.at[:, pl.ds(col0, CPT)], red_v)

    def _red(j, carry):
        acc = red_v[0, pl.ds(j * 16, 16)]
        for r in range(1, NS):
            acc = acc + red_v[r, pl.ds(j * 16, 16)]
        res_v[pl.ds(j * 16, 16)] = acc
        return carry
    lax.fori_loop(0, CPT // 16, _red, 0)
    pltpu.sync_copy(res_v, pdeg_hbm.at[pl.ds(c * NP + col0, CPT)])


_deg_kernel = pl.kernel(
    _deg_body,
    out_type=jax.ShapeDtypeStruct((2 * NP,), jnp.float32),
    mesh=_sc_mesh(),
    compiler_params=pltpu.CompilerParams(needs_layout_passes=False),
    scratch_types=[
        pltpu.VMEM((EPT + 16,), jnp.int32),
        pltpu.VMEM((NP,), jnp.float32),
        pltpu.VMEM_SHARED((NS, NP), jnp.float32),
        pltpu.VMEM((NS, CPT), jnp.float32),
        pltpu.VMEM((CPT,), jnp.float32),
    ],
)


def _mm_body(p0_ref, p1_ref, x_ref, w_ref, z_ref):
    deg = p0_ref[...] + p1_ref[...]
    dis = jnp.where(deg > 0, lax.rsqrt(deg), 0.0)
    xw = jnp.dot(x_ref[...], w_ref[...], preferred_element_type=jnp.float32)
    z_ref[...] = xw * dis


def _mm_call(p0, p1, x, weight):
    return pl.pallas_call(
        _mm_body,
        grid=(NRB, NC),
        in_specs=[
            pl.BlockSpec((RB, 1), lambda i, c: (i, 0)),
            pl.BlockSpec((RB, 1), lambda i, c: (i, 0)),
            pl.BlockSpec((RB, DIN), lambda i, c: (i, 0)),
            pl.BlockSpec((DIN, DH), lambda i, c: (0, c)),
        ],
        out_specs=pl.BlockSpec((RB, DH), lambda i, c: (c * NRB + i, 0)),
        out_shape=jax.ShapeDtypeStruct((NC * NN, DH), jnp.float32),
    )(p0, p1, x, weight)


def _agg_body(src_hbm, dst_hbm, z_hbm, zrow_hbm, agg_hbm,
              slab_sh, idx_g, idx_s, rows_v, drain_v, sem):
    c = lax.axis_index("c")
    s = lax.axis_index("s")
    pltpu.sync_copy(zrow_hbm, slab_sh.at[pl.ds(s * RPT, RPT)])
    plsc.subcore_barrier()
    ebase = s * EPS
    coff = c * NN

    def _step(it, carry):
        off = ebase + it * CH
        pltpu.sync_copy(dst_hbm.at[pl.ds(off, CH)], idx_g.at[0])
        pltpu.sync_copy(src_hbm.at[pl.ds(off, CH)], idx_s.at[0])
        for k in range(CH // 16):
            v = idx_g[0, pl.ds(k * 16, 16)]
            idx_g[0, pl.ds(k * 16, 16)] = v + coff
        pltpu.async_copy(z_hbm.at[idx_g.at[0]], rows_v.at[0], sem).wait()
        pltpu.sync_copy(rows_v.at[0], slab_sh.at[idx_s.at[0]], add=True)
        return carry
    lax.fori_loop(0, EPS // CH, _step, 0)
    plsc.subcore_barrier()

    for k in range(RPT // DRC):
        r0 = s * RPT + k * DRC
        pltpu.sync_copy(slab_sh.at[pl.ds(r0, DRC)], drain_v)
        pltpu.sync_copy(drain_v, agg_hbm.at[c, pl.ds(r0, DRC)])


_agg_kernel = pl.kernel(
    _agg_body,
    out_type=jax.ShapeDtypeStruct((NC, NP, DH), jnp.float32),
    mesh=_sc_mesh(),
    compiler_params=pltpu.CompilerParams(needs_layout_passes=False),
    scratch_types=[
        pltpu.VMEM_SHARED((NP, DH), jnp.float32),
        pltpu.VMEM((1, CH), jnp.int32),
        pltpu.VMEM((1, CH), jnp.int32),
        pltpu.VMEM((1, CH, DH), jnp.float32),
        pltpu.VMEM((DRC, DH), jnp.float32),
        pltpu.SemaphoreType.DMA,
    ],
)


def _fin_body(p0_ref, p1_ref, bias_ref, agg_ref, out_ref):
    deg = p0_ref[...] + p1_ref[...]
    dis = jnp.where(deg > 0, lax.rsqrt(deg), 0.0)
    out_ref[...] = agg_ref[0] * dis + bias_ref[...]


def _fin_call(p0, p1, bias2d, agg):
    return pl.pallas_call(
        _fin_body,
        grid=(NRB, NC),
        in_specs=[
            pl.BlockSpec((RB, 1), lambda i, c: (i, 0)),
            pl.BlockSpec((RB, 1), lambda i, c: (i, 0)),
            pl.BlockSpec((1, DH), lambda i, c: (0, c)),
            pl.BlockSpec((1, RB, DH), lambda i, c: (c, i, 0)),
        ],
        out_specs=pl.BlockSpec((RB, DH), lambda i, c: (i, c)),
        out_shape=jax.ShapeDtypeStruct((NN, DOUT), jnp.float32),
    )(p0, p1, bias2d, agg)


def kernel(x, edge_index, weight, bias):
    edge_index = edge_index.astype(jnp.int32)
    src = edge_index[0]
    dst = edge_index[1]
    pdeg = _deg_kernel(src)
    p0 = pdeg[:NP].reshape(NP, 1)
    p1 = pdeg[NP:].reshape(NP, 1)
    z = _mm_call(p0, p1, x, weight)
    zrows = jnp.zeros((RPT, DH), jnp.float32)
    agg = _agg_kernel(src, dst, z, zrows)
    return _fin_call(p0, p1, bias.reshape(1, DOUT), agg)


# speedup vs baseline: 14.1558x; 1.5938x over previous
import functools

import jax
import jax.numpy as jnp
from jax import lax
from jax.experimental import pallas as pl
from jax.experimental.pallas import tpu as pltpu
from jax.experimental.pallas import tpu_sc as plsc

NN = 10000
EE = 160000
DIN = 256
DOUT = 256
DH = 128
NC = 2
NS = 16
NP = 10240
CPT = NP // NS
EPT = EE // (NC * NS)
EPS = EE // NS
CH = 40
RPT = NP // NS
DRC = 128
RB = 2000
NRB = NN // RB


def _sc_mesh():
    return plsc.VectorSubcoreMesh(
        core_axis_name="c", subcore_axis_name="s",
        num_cores=NC, num_subcores=NS)


def _deg_body(src_hbm, pdeg_hbm, src_v, deg_v, stage_sh, red_v, res_v):
    c = lax.axis_index("c")
    s = lax.axis_index("s")
    wid = c * NS + s
    zeros16 = jnp.zeros((16,), jnp.float32)
    ones16 = jnp.ones((16,), jnp.float32)
    lane = lax.iota(jnp.int32, 16)

    def _zero(i, carry):
        deg_v[pl.ds(i * 16, 16)] = zeros16
        return carry
    lax.fori_loop(0, NP // 16, _zero, 0)

    pltpu.sync_copy(src_hbm.at[pl.ds(wid * EPT, EPT)],
                    src_v.at[pl.ds(0, EPT)])

    def _scat(i, carry):
        base = i * 16
        idx = src_v[pl.ds(base, 16)]
        idx = jnp.minimum(jnp.maximum(idx, 0), NP - 1)
        m = lane < (EPT - base)
        plsc.addupdate_scatter(deg_v, [idx], ones16, mask=m)
        return carry
    lax.fori_loop(0, (EPT + 15) // 16, _scat, 0)

    pltpu.sync_copy(deg_v, stage_sh.at[s])
    plsc.subcore_barrier()
    col0 = s * CPT
    pltpu.sync_copy(stage_sh.at[:, pl.ds(col0, CPT)], red_v)

    def _red(j, carry):
        acc = red_v[0, pl.ds(j * 16, 16)]
        for r in range(1, NS):
            acc = acc + red_v[r, pl.ds(j * 16, 16)]
        res_v[pl.ds(j * 16, 16)] = acc
        return carry
    lax.fori_loop(0, CPT // 16, _red, 0)
    pltpu.sync_copy(res_v, pdeg_hbm.at[pl.ds(c * NP + col0, CPT)])


_deg_kernel = pl.kernel(
    _deg_body,
    out_type=jax.ShapeDtypeStruct((2 * NP,), jnp.float32),
    mesh=_sc_mesh(),
    compiler_params=pltpu.CompilerParams(needs_layout_passes=False),
    scratch_types=[
        pltpu.VMEM((EPT + 16,), jnp.int32),
        pltpu.VMEM((NP,), jnp.float32),
        pltpu.VMEM_SHARED((NS, NP), jnp.float32),
        pltpu.VMEM((NS, CPT), jnp.float32),
        pltpu.VMEM((CPT,), jnp.float32),
    ],
)


def _mm_body(p0_ref, p1_ref, x_ref, w_ref, z0_ref, z1_ref):
    deg = p0_ref[...] + p1_ref[...]
    dis = jnp.where(deg > 0, lax.rsqrt(deg), 0.0)
    xw = jnp.dot(x_ref[...], w_ref[...], preferred_element_type=jnp.float32)
    z0_ref[...] = xw[:, :DH] * dis
    z1_ref[...] = xw[:, DH:] * dis


def _mm_call(p0, p1, x, weight):
    return pl.pallas_call(
        _mm_body,
        grid=(NRB,),
        in_specs=[
            pl.BlockSpec((RB, 1), lambda i: (i, 0)),
            pl.BlockSpec((RB, 1), lambda i: (i, 0)),
            pl.BlockSpec((RB, DIN), lambda i: (i, 0)),
            pl.BlockSpec((DIN, DOUT), lambda i: (0, 0)),
        ],
        out_specs=[
            pl.BlockSpec((RB, DH), lambda i: (i, 0)),
            pl.BlockSpec((RB, DH), lambda i: (i, 0)),
        ],
        out_shape=[
            jax.ShapeDtypeStruct((NN, DH), jnp.float32),
            jax.ShapeDtypeStruct((NN, DH), jnp.float32),
        ],
    )(p0, p1, x, weight)


NCH = EPS // CH
GSZ = 5
NGRP = NCH // GSZ
NZC = NN // CH


def _agg_body(src4_hbm, dst4_hbm, z0_hbm, z1_hbm, agg_hbm,
              slab_sh, idx_d2, idx_s2, rows_v, sem_g, sem_s):
    c = lax.axis_index("c")
    s = lax.axis_index("s")
    zeros16 = jnp.zeros((16,), jnp.float32)

    def _zrow(r, carry):
        for k in range(DH // 16):
            rows_v[0, r, pl.ds(k * 16, 16)] = zeros16
        return carry
    lax.fori_loop(0, CH, _zrow, 0)
    for k in range((NZC + NS - 1) // NS):
        cid = k * NS + s

        @pl.when(cid < NZC)
        def _():
            pltpu.sync_copy(rows_v.at[0], slab_sh.at[pl.ds(cid * CH, CH)])

    plsc.subcore_barrier()

    def _run(z_ref):
        def _group(g, carry):
            pltpu.sync_copy(dst4_hbm.at[s, g], idx_d2)
            pltpu.sync_copy(src4_hbm.at[s, g], idx_s2)
            gets = []
            for b in range(GSZ):
                gets.append(pltpu.async_copy(
                    z_ref.at[idx_d2.at[b]], rows_v.at[b], sem_g))
            puts = []
            for b in range(GSZ):
                gets[b].wait()
                puts.append(pltpu.async_copy(
                    rows_v.at[b], slab_sh.at[idx_s2.at[b]],
                    sem_s, add=True))
            for p in puts:
                p.wait()
            return carry
        lax.fori_loop(0, NGRP, _group, 0)

    @pl.when(c == 0)
    def _():
        _run(z0_hbm)

    @pl.when(c == 1)
    def _():
        _run(z1_hbm)

    plsc.subcore_barrier()

    for k in range((NZC + NS - 1) // NS):
        cid = k * NS + s

        @pl.when(cid < NZC)
        def _():
            pltpu.sync_copy(slab_sh.at[pl.ds(cid * CH, CH)], rows_v.at[0])
            pltpu.sync_copy(rows_v.at[0], agg_hbm.at[c, pl.ds(cid * CH, CH)])


_agg_kernel = pl.kernel(
    _agg_body,
    out_type=jax.ShapeDtypeStruct((NC, NN, DH), jnp.float32),
    mesh=_sc_mesh(),
    compiler_params=pltpu.CompilerParams(needs_layout_passes=False),
    scratch_types=[
        pltpu.VMEM_SHARED((NN, DH), jnp.float32),
        pltpu.VMEM((GSZ, CH), jnp.int32),
        pltpu.VMEM((GSZ, CH), jnp.int32),
        pltpu.VMEM((GSZ, CH, DH), jnp.float32),
        pltpu.SemaphoreType.DMA,
        pltpu.SemaphoreType.DMA,
    ],
)


def _fin_body(p0_ref, p1_ref, bias_ref, agg_ref, out_ref):
    deg = p0_ref[...] + p1_ref[...]
    dis = jnp.where(deg > 0, lax.rsqrt(deg), 0.0)
    out_ref[...] = agg_ref[0] * dis + bias_ref[...]


def _fin_call(p0, p1, bias2d, agg):
    return pl.pallas_call(
        _fin_body,
        grid=(NRB, NC),
        in_specs=[
            pl.BlockSpec((RB, 1), lambda i, c: (i, 0)),
            pl.BlockSpec((RB, 1), lambda i, c: (i, 0)),
            pl.BlockSpec((1, DH), lambda i, c: (0, c)),
            pl.BlockSpec((1, RB, DH), lambda i, c: (c, i, 0)),
        ],
        out_specs=pl.BlockSpec((RB, DH), lambda i, c: (i, c)),
        out_shape=jax.ShapeDtypeStruct((NN, DOUT), jnp.float32),
    )(p0, p1, bias2d, agg)


def kernel(x, edge_index, weight, bias):
    edge_index = edge_index.astype(jnp.int32)
    src = edge_index[0]
    dst = edge_index[1]
    pdeg = _deg_kernel(src)
    p0 = pdeg[:NP].reshape(NP, 1)
    p1 = pdeg[NP:].reshape(NP, 1)
    z0, z1 = _mm_call(p0, p1, x, weight)
    src4 = src.reshape(NS, NGRP, GSZ, CH)
    dst4 = dst.reshape(NS, NGRP, GSZ, CH)
    agg = _agg_kernel(src4, dst4, z0, z1)
    return _fin_call(p0, p1, bias.reshape(1, DOUT), agg)


# speedup vs baseline: 18.6099x; 1.3146x over previous
import functools

import jax
import jax.numpy as jnp
from jax import lax
from jax.experimental import pallas as pl
from jax.experimental.pallas import tpu as pltpu
from jax.experimental.pallas import tpu_sc as plsc

NN = 10000
EE = 160000
DIN = 256
DOUT = 256
DH = 128
NC = 2
NS = 16
NP = 10240
CPT = NP // NS
EPT = EE // (NC * NS)
EPS = EE // NS
CH = 40
RPT = NP // NS
DRC = 128
RB = 2000
NRB = NN // RB


def _sc_mesh():
    return plsc.VectorSubcoreMesh(
        core_axis_name="c", subcore_axis_name="s",
        num_cores=NC, num_subcores=NS)


def _deg_body(src_hbm, pdeg_hbm, src_v, deg_v, stage_sh, red_v, res_v):
    c = lax.axis_index("c")
    s = lax.axis_index("s")
    wid = c * NS + s
    zeros16 = jnp.zeros((16,), jnp.float32)
    ones16 = jnp.ones((16,), jnp.float32)
    lane = lax.iota(jnp.int32, 16)

    def _zero(i, carry):
        deg_v[pl.ds(i * 16, 16)] = zeros16
        return carry
    lax.fori_loop(0, NP // 16, _zero, 0)

    pltpu.sync_copy(src_hbm.at[pl.ds(wid * EPT, EPT)],
                    src_v.at[pl.ds(0, EPT)])

    def _scat(i, carry):
        base = i * 16
        idx = src_v[pl.ds(base, 16)]
        idx = jnp.minimum(jnp.maximum(idx, 0), NP - 1)
        m = lane < (EPT - base)
        plsc.addupdate_scatter(deg_v, [idx], ones16, mask=m)
        return carry
    lax.fori_loop(0, (EPT + 15) // 16, _scat, 0)

    pltpu.sync_copy(deg_v, stage_sh.at[s])
    plsc.subcore_barrier()
    col0 = s * CPT
    pltpu.sync_copy(stage_sh.at[:, pl.ds(col0, CPT)], red_v)

    def _red(j, carry):
        acc = red_v[0, pl.ds(j * 16, 16)]
        for r in range(1, NS):
            acc = acc + red_v[r, pl.ds(j * 16, 16)]
        res_v[pl.ds(j * 16, 16)] = acc
        return carry
    lax.fori_loop(0, CPT // 16, _red, 0)
    pltpu.sync_copy(res_v, pdeg_hbm.at[pl.ds(c * NP + col0, CPT)])


_deg_kernel = pl.kernel(
    _deg_body,
    out_type=jax.ShapeDtypeStruct((2 * NP,), jnp.float32),
    mesh=_sc_mesh(),
    compiler_params=pltpu.CompilerParams(needs_layout_passes=False),
    scratch_types=[
        pltpu.VMEM((EPT + 16,), jnp.int32),
        pltpu.VMEM((NP,), jnp.float32),
        pltpu.VMEM_SHARED((NS, NP), jnp.float32),
        pltpu.VMEM((NS, CPT), jnp.float32),
        pltpu.VMEM((CPT,), jnp.float32),
    ],
)


def _mm_body(p0_ref, p1_ref, x_ref, w_ref, z0_ref, z1_ref):
    deg = p0_ref[...] + p1_ref[...]
    dis = jnp.where(deg > 0, lax.rsqrt(deg), 0.0)
    xw = jnp.dot(x_ref[...], w_ref[...], preferred_element_type=jnp.float32)
    z0_ref[...] = xw[:, :DH] * dis
    z1_ref[...] = xw[:, DH:] * dis


def _mm_call(p0, p1, x, weight):
    return pl.pallas_call(
        _mm_body,
        grid=(NRB,),
        in_specs=[
            pl.BlockSpec((RB, 1), lambda i: (i, 0)),
            pl.BlockSpec((RB, 1), lambda i: (i, 0)),
            pl.BlockSpec((RB, DIN), lambda i: (i, 0)),
            pl.BlockSpec((DIN, DOUT), lambda i: (0, 0)),
        ],
        out_specs=[
            pl.BlockSpec((RB, DH), lambda i: (i, 0)),
            pl.BlockSpec((RB, DH), lambda i: (i, 0)),
        ],
        out_shape=[
            jax.ShapeDtypeStruct((NN, DH), jnp.float32),
            jax.ShapeDtypeStruct((NN, DH), jnp.float32),
        ],
    )(p0, p1, x, weight)


NCH = EPS // CH
GSZ = 5
NGRP = NCH // GSZ
NZC = NN // CH


def _agg_body(src4_hbm, dst4_hbm, z0_hbm, z1_hbm, agg_hbm,
              slab_sh, idx_d2, idx_s2, rows_v, sem_g, sem_s, sem_i):
    c = lax.axis_index("c")
    s = lax.axis_index("s")
    zeros16 = jnp.zeros((16,), jnp.float32)

    def _zrow(r, carry):
        for k in range(DH // 16):
            rows_v[0, r, pl.ds(k * 16, 16)] = zeros16
        return carry
    lax.fori_loop(0, CH, _zrow, 0)
    for k in range((NZC + NS - 1) // NS):
        cid = k * NS + s

        @pl.when(cid < NZC)
        def _():
            pltpu.sync_copy(rows_v.at[0], slab_sh.at[pl.ds(cid * CH, CH)])

    plsc.subcore_barrier()

    def _run(z_ref):
        pltpu.sync_copy(dst4_hbm.at[s, 0], idx_d2.at[0])
        pltpu.sync_copy(src4_hbm.at[s, 0], idx_s2.at[0])

        def _group(h, carry):
            for p in range(2):
                gg = h * 2 + p

                for b in range(GSZ):
                    @pl.when(gg > 0)
                    def _():
                        pltpu.make_async_copy(
                            rows_v.at[b], slab_sh.at[idx_s2.at[p, b]],
                            sem_s).wait()
                    pltpu.async_copy(
                        z_ref.at[idx_d2.at[p, b]], rows_v.at[b], sem_g)

                @pl.when(gg + 1 < NGRP)
                def _():
                    pltpu.async_copy(dst4_hbm.at[s, gg + 1],
                                     idx_d2.at[1 - p], sem_i)
                    pltpu.async_copy(src4_hbm.at[s, gg + 1],
                                     idx_s2.at[1 - p], sem_i)

                for b in range(GSZ):
                    pltpu.make_async_copy(
                        z_ref.at[idx_d2.at[p, b]], rows_v.at[b], sem_g).wait()
                    pltpu.async_copy(
                        rows_v.at[b], slab_sh.at[idx_s2.at[p, b]],
                        sem_s, add=True)

                @pl.when(gg + 1 < NGRP)
                def _():
                    pltpu.make_async_copy(dst4_hbm.at[s, gg + 1],
                                          idx_d2.at[1 - p], sem_i).wait()
                    pltpu.make_async_copy(src4_hbm.at[s, gg + 1],
                                          idx_s2.at[1 - p], sem_i).wait()
            return carry
        lax.fori_loop(0, NGRP // 2, _group, 0)
        for b in range(GSZ):
            pltpu.make_async_copy(
                rows_v.at[b], slab_sh.at[idx_s2.at[1, b]], sem_s).wait()

    @pl.when(c == 0)
    def _():
        _run(z0_hbm)

    @pl.when(c == 1)
    def _():
        _run(z1_hbm)

    plsc.subcore_barrier()

    for k in range((NZC + NS - 1) // NS):
        cid = k * NS + s

        @pl.when(cid < NZC)
        def _():
            pltpu.sync_copy(slab_sh.at[pl.ds(cid * CH, CH)], rows_v.at[0])
            pltpu.sync_copy(rows_v.at[0], agg_hbm.at[c, pl.ds(cid * CH, CH)])


_agg_kernel = pl.kernel(
    _agg_body,
    out_type=jax.ShapeDtypeStruct((NC, NN, DH), jnp.float32),
    mesh=_sc_mesh(),
    compiler_params=pltpu.CompilerParams(needs_layout_passes=False),
    scratch_types=[
        pltpu.VMEM_SHARED((NN, DH), jnp.float32),
        pltpu.VMEM((2, GSZ, CH), jnp.int32),
        pltpu.VMEM((2, GSZ, CH), jnp.int32),
        pltpu.VMEM((GSZ, CH, DH), jnp.float32),
        pltpu.SemaphoreType.DMA,
        pltpu.SemaphoreType.DMA,
        pltpu.SemaphoreType.DMA,
    ],
)


def _fin_body(p0_ref, p1_ref, bias_ref, agg_ref, out_ref):
    deg = p0_ref[...] + p1_ref[...]
    dis = jnp.where(deg > 0, lax.rsqrt(deg), 0.0)
    out_ref[...] = agg_ref[0] * dis + bias_ref[...]


def _fin_call(p0, p1, bias2d, agg):
    return pl.pallas_call(
        _fin_body,
        grid=(NRB, NC),
        in_specs=[
            pl.BlockSpec((RB, 1), lambda i, c: (i, 0)),
            pl.BlockSpec((RB, 1), lambda i, c: (i, 0)),
            pl.BlockSpec((1, DH), lambda i, c: (0, c)),
            pl.BlockSpec((1, RB, DH), lambda i, c: (c, i, 0)),
        ],
        out_specs=pl.BlockSpec((RB, DH), lambda i, c: (i, c)),
        out_shape=jax.ShapeDtypeStruct((NN, DOUT), jnp.float32),
    )(p0, p1, bias2d, agg)


def kernel(x, edge_index, weight, bias):
    edge_index = edge_index.astype(jnp.int32)
    src = edge_index[0]
    dst = edge_index[1]
    pdeg = _deg_kernel(src)
    p0 = pdeg[:NP].reshape(NP, 1)
    p1 = pdeg[NP:].reshape(NP, 1)
    z0, z1 = _mm_call(p0, p1, x, weight)
    src4 = src.reshape(NS, NGRP, GSZ, CH)
    dst4 = dst.reshape(NS, NGRP, GSZ, CH)
    agg = _agg_kernel(src4, dst4, z0, z1)
    return _fin_call(p0, p1, bias.reshape(1, DOUT), agg)


# speedup vs baseline: 19.4307x; 1.0441x over previous
import functools

import jax
import jax.numpy as jnp
from jax import lax
from jax.experimental import pallas as pl
from jax.experimental.pallas import tpu as pltpu
from jax.experimental.pallas import tpu_sc as plsc

NN = 10000
EE = 160000
DIN = 256
DOUT = 256
DH = 128
NC = 2
NS = 16
NP = 10240
CPT = NP // NS
EPT = EE // (NC * NS)
EPS = EE // NS
CH = 40
RPT = NP // NS
DRC = 128
RB = 2000
NRB = NN // RB


def _sc_mesh():
    return plsc.VectorSubcoreMesh(
        core_axis_name="c", subcore_axis_name="s",
        num_cores=NC, num_subcores=NS)


def _deg_body(src_hbm, pdeg_hbm, src_v, deg_v, stage_sh, red_v, res_v):
    c = lax.axis_index("c")
    s = lax.axis_index("s")
    wid = c * NS + s
    zeros16 = jnp.zeros((16,), jnp.float32)
    ones16 = jnp.ones((16,), jnp.float32)
    lane = lax.iota(jnp.int32, 16)

    def _zero(i, carry):
        deg_v[pl.ds(i * 16, 16)] = zeros16
        return carry
    lax.fori_loop(0, NP // 16, _zero, 0)

    pltpu.sync_copy(src_hbm.at[pl.ds(wid * EPT, EPT)],
                    src_v.at[pl.ds(0, EPT)])

    def _scat(i, carry):
        base = i * 16
        idx = src_v[pl.ds(base, 16)]
        idx = jnp.minimum(jnp.maximum(idx, 0), NP - 1)
        m = lane < (EPT - base)
        plsc.addupdate_scatter(deg_v, [idx], ones16, mask=m)
        return carry
    lax.fori_loop(0, (EPT + 15) // 16, _scat, 0)

    pltpu.sync_copy(deg_v, stage_sh.at[s])
    plsc.subcore_barrier()
    col0 = s * CPT
    pltpu.sync_copy(stage_sh.at[:, pl.ds(col0, CPT)], red_v)

    def _red(j, carry):
        acc = red_v[0, pl.ds(j * 16, 16)]
        for r in range(1, NS):
            acc = acc + red_v[r, pl.ds(j * 16, 16)]
        res_v[pl.ds(j * 16, 16)] = acc
        return carry
    lax.fori_loop(0, CPT // 16, _red, 0)
    pltpu.sync_copy(res_v, pdeg_hbm.at[pl.ds(c * NP + col0, CPT)])


_deg_kernel = pl.kernel(
    _deg_body,
    out_type=jax.ShapeDtypeStruct((2 * NP,), jnp.float32),
    mesh=_sc_mesh(),
    compiler_params=pltpu.CompilerParams(needs_layout_passes=False),
    scratch_types=[
        pltpu.VMEM((EPT + 16,), jnp.int32),
        pltpu.VMEM((NP,), jnp.float32),
        pltpu.VMEM_SHARED((NS, NP), jnp.float32),
        pltpu.VMEM((NS, CPT), jnp.float32),
        pltpu.VMEM((CPT,), jnp.float32),
    ],
)


def _mm_body(p0_ref, p1_ref, x_ref, w_ref, z0_ref, z1_ref):
    deg = p0_ref[...] + p1_ref[...]
    dis = jnp.where(deg > 0, lax.rsqrt(deg), 0.0)
    xw = jnp.dot(x_ref[...], w_ref[...], preferred_element_type=jnp.float32)
    z0_ref[...] = xw[:, :DH] * dis
    z1_ref[...] = xw[:, DH:] * dis


def _mm_call(p0, p1, x, weight):
    return pl.pallas_call(
        _mm_body,
        grid=(NRB,),
        in_specs=[
            pl.BlockSpec((RB, 1), lambda i: (i, 0)),
            pl.BlockSpec((RB, 1), lambda i: (i, 0)),
            pl.BlockSpec((RB, DIN), lambda i: (i, 0)),
            pl.BlockSpec((DIN, DOUT), lambda i: (0, 0)),
        ],
        out_specs=[
            pl.BlockSpec((RB, DH), lambda i: (i, 0)),
            pl.BlockSpec((RB, DH), lambda i: (i, 0)),
        ],
        out_shape=[
            jax.ShapeDtypeStruct((NN, DH), jnp.float32),
            jax.ShapeDtypeStruct((NN, DH), jnp.float32),
        ],
    )(p0, p1, x, weight)


NCH = EPS // CH
GSZ = 5
NGRP = NCH // GSZ
NZC = NN // CH


def _agg_body(src4_hbm, dst4_hbm, z0_hbm, z1_hbm, agg_hbm,
              slab_sh, idx_d2, idx_s2, rows_v, sem_g, sem_s, sem_i):
    c = lax.axis_index("c")
    s = lax.axis_index("s")
    zeros16 = jnp.zeros((16,), jnp.float32)

    def _zrow(r, carry):
        for k in range(DH // 16):
            rows_v[0, r, pl.ds(k * 16, 16)] = zeros16
        return carry
    lax.fori_loop(0, CH, _zrow, 0)
    for k in range((NZC + NS - 1) // NS):
        cid = k * NS + s

        @pl.when(cid < NZC)
        def _():
            pltpu.async_copy(rows_v.at[0], slab_sh.at[pl.ds(cid * CH, CH)],
                             sem_g)
    for k in range((NZC + NS - 1) // NS):
        cid = k * NS + s

        @pl.when(cid < NZC)
        def _():
            pltpu.make_async_copy(rows_v.at[0],
                                  slab_sh.at[pl.ds(cid * CH, CH)],
                                  sem_g).wait()

    plsc.subcore_barrier()

    def _run(z_ref):
        pltpu.sync_copy(dst4_hbm.at[s, 0], idx_d2.at[0])
        pltpu.sync_copy(src4_hbm.at[s, 0], idx_s2.at[0])

        def _group(h, carry):
            for p in range(2):
                gg = h * 2 + p

                for b in range(GSZ):
                    @pl.when(gg > 0)
                    def _():
                        pltpu.make_async_copy(
                            rows_v.at[b], slab_sh.at[idx_s2.at[p, b]],
                            sem_s).wait()
                    pltpu.async_copy(
                        z_ref.at[idx_d2.at[p, b]], rows_v.at[b], sem_g)

                @pl.when(gg + 1 < NGRP)
                def _():
                    pltpu.async_copy(dst4_hbm.at[s, gg + 1],
                                     idx_d2.at[1 - p], sem_i)
                    pltpu.async_copy(src4_hbm.at[s, gg + 1],
                                     idx_s2.at[1 - p], sem_i)

                for b in range(GSZ):
                    pltpu.make_async_copy(
                        z_ref.at[idx_d2.at[p, b]], rows_v.at[b], sem_g).wait()
                    pltpu.async_copy(
                        rows_v.at[b], slab_sh.at[idx_s2.at[p, b]],
                        sem_s, add=True)

                @pl.when(gg + 1 < NGRP)
                def _():
                    pltpu.make_async_copy(dst4_hbm.at[s, gg + 1],
                                          idx_d2.at[1 - p], sem_i).wait()
                    pltpu.make_async_copy(src4_hbm.at[s, gg + 1],
                                          idx_s2.at[1 - p], sem_i).wait()
            return carry
        lax.fori_loop(0, NGRP // 2, _group, 0)
        for b in range(GSZ):
            pltpu.make_async_copy(
                rows_v.at[b], slab_sh.at[idx_s2.at[1, b]], sem_s).wait()

    @pl.when(c == 0)
    def _():
        _run(z0_hbm)

    @pl.when(c == 1)
    def _():
        _run(z1_hbm)

    plsc.subcore_barrier()

    nk = (NZC + NS - 1) // NS
    for k in range(nk):
        cid = k * NS + s
        b = k % GSZ

        @pl.when(cid < NZC)
        def _():
            if k >= GSZ:
                pltpu.make_async_copy(
                    rows_v.at[b], agg_hbm.at[c, pl.ds(cid * CH, CH)],
                    sem_s).wait()
            pltpu.async_copy(slab_sh.at[pl.ds(cid * CH, CH)], rows_v.at[b],
                             sem_g).wait()
            pltpu.async_copy(rows_v.at[b], agg_hbm.at[c, pl.ds(cid * CH, CH)],
                             sem_s)
    for b in range(GSZ):
        pltpu.make_async_copy(rows_v.at[b],
                              agg_hbm.at[c, pl.ds(s * CH, CH)],
                              sem_s).wait()


_agg_kernel = pl.kernel(
    _agg_body,
    out_type=jax.ShapeDtypeStruct((NC, NN, DH), jnp.float32),
    mesh=_sc_mesh(),
    compiler_params=pltpu.CompilerParams(needs_layout_passes=False),
    scratch_types=[
        pltpu.VMEM_SHARED((NN, DH), jnp.float32),
        pltpu.VMEM((2, GSZ, CH), jnp.int32),
        pltpu.VMEM((2, GSZ, CH), jnp.int32),
        pltpu.VMEM((GSZ, CH, DH), jnp.float32),
        pltpu.SemaphoreType.DMA,
        pltpu.SemaphoreType.DMA,
        pltpu.SemaphoreType.DMA,
    ],
)


def _fin_body(p0_ref, p1_ref, bias_ref, agg_ref, out_ref):
    deg = p0_ref[...] + p1_ref[...]
    dis = jnp.where(deg > 0, lax.rsqrt(deg), 0.0)
    out_ref[:, :DH] = agg_ref[0] * dis + bias_ref[:, :DH]
    out_ref[:, DH:] = agg_ref[1] * dis + bias_ref[:, DH:]


def _fin_call(p0, p1, bias2d, agg):
    return pl.pallas_call(
        _fin_body,
        grid=(NRB,),
        in_specs=[
            pl.BlockSpec((RB, 1), lambda i: (i, 0)),
            pl.BlockSpec((RB, 1), lambda i: (i, 0)),
            pl.BlockSpec((1, DOUT), lambda i: (0, 0)),
            pl.BlockSpec((NC, RB, DH), lambda i: (0, i, 0)),
        ],
        out_specs=pl.BlockSpec((RB, DOUT), lambda i: (i, 0)),
        out_shape=jax.ShapeDtypeStruct((NN, DOUT), jnp.float32),
    )(p0, p1, bias2d, agg)


def kernel(x, edge_index, weight, bias):
    edge_index = edge_index.astype(jnp.int32)
    src = edge_index[0]
    dst = edge_index[1]
    pdeg = _deg_kernel(src)
    p0 = pdeg[:NP].reshape(NP, 1)
    p1 = pdeg[NP:].reshape(NP, 1)
    z0, z1 = _mm_call(p0, p1, x, weight)
    src4 = src.reshape(NS, NGRP, GSZ, CH)
    dst4 = dst.reshape(NS, NGRP, GSZ, CH)
    agg = _agg_kernel(src4, dst4, z0, z1)
    return _fin_call(p0, p1, bias.reshape(1, DOUT), agg)


# speedup vs baseline: 20.3325x; 1.0464x over previous
import functools

import jax
import jax.numpy as jnp
from jax import lax
from jax.experimental import pallas as pl
from jax.experimental.pallas import tpu as pltpu
from jax.experimental.pallas import tpu_sc as plsc

NN = 10000
EE = 160000
DIN = 256
DOUT = 256
DH = 128
NC = 2
NS = 16
NP = 10240
CPT = NP // NS
EPT = EE // (NC * NS)
EPS = EE // NS
CH = 40
RPT = NP // NS
DRC = 128
RB = 2000
NRB = NN // RB


def _sc_mesh():
    return plsc.VectorSubcoreMesh(
        core_axis_name="c", subcore_axis_name="s",
        num_cores=NC, num_subcores=NS)


def _deg_body(edge_hbm, pdeg_hbm, src_v, deg_v, stage_sh, red_v, res_v):
    c = lax.axis_index("c")
    s = lax.axis_index("s")
    wid = c * NS + s
    zeros16 = jnp.zeros((16,), jnp.float32)
    ones16 = jnp.ones((16,), jnp.float32)
    lane = lax.iota(jnp.int32, 16)

    def _zero(i, carry):
        deg_v[pl.ds(i * 16, 16)] = zeros16
        return carry
    lax.fori_loop(0, NP // 16, _zero, 0)

    aligned = wid * EPT // 128 * 128
    rem = wid * EPT - aligned
    pltpu.sync_copy(edge_hbm.at[:, pl.ds(aligned, EPT + 120)],
                    src_v.at[:, pl.ds(0, EPT + 120)])

    def _scat(i, carry):
        base = i * 16
        idx = src_v[0, pl.ds(rem + base, 16)]
        idx = jnp.minimum(jnp.maximum(idx, 0), NP - 1)
        m = lane < (EPT - base)
        plsc.addupdate_scatter(deg_v, [idx], ones16, mask=m)
        return carry
    lax.fori_loop(0, (EPT + 15) // 16, _scat, 0)

    pltpu.sync_copy(deg_v, stage_sh.at[s])
    plsc.subcore_barrier()
    col0 = s * CPT
    pltpu.sync_copy(stage_sh.at[:, pl.ds(col0, CPT)], red_v)

    def _red(j, carry):
        acc = red_v[0, pl.ds(j * 16, 16)]
        for r in range(1, NS):
            acc = acc + red_v[r, pl.ds(j * 16, 16)]
        res_v[pl.ds(j * 16, 16)] = acc
        return carry
    lax.fori_loop(0, CPT // 16, _red, 0)
    pltpu.sync_copy(res_v, pdeg_hbm.at[pl.ds(c * NP + col0, CPT)])


_deg_kernel = pl.kernel(
    _deg_body,
    out_type=jax.ShapeDtypeStruct((2 * NP,), jnp.float32),
    mesh=_sc_mesh(),
    compiler_params=pltpu.CompilerParams(needs_layout_passes=False),
    scratch_types=[
        pltpu.VMEM((2, EPT + 136), jnp.int32),
        pltpu.VMEM((NP,), jnp.float32),
        pltpu.VMEM_SHARED((NS, NP), jnp.float32),
        pltpu.VMEM((NS, CPT), jnp.float32),
        pltpu.VMEM((CPT,), jnp.float32),
    ],
)


def _mm_body(p01_ref, x_ref, w_ref, z0_ref, z1_ref):
    pd = p01_ref[...]
    deg = pd[:, :1] + pd[:, 1:]
    dis = jnp.where(deg > 0, lax.rsqrt(deg), 0.0)
    xw = jnp.dot(x_ref[...], w_ref[...], preferred_element_type=jnp.float32)
    z0_ref[...] = xw[:, :DH] * dis
    z1_ref[...] = xw[:, DH:] * dis


def _mm_call(p01, x, weight):
    return pl.pallas_call(
        _mm_body,
        grid=(NRB,),
        in_specs=[
            pl.BlockSpec((RB, 2), lambda i: (i, 0)),
            pl.BlockSpec((RB, DIN), lambda i: (i, 0)),
            pl.BlockSpec((DIN, DOUT), lambda i: (0, 0)),
        ],
        out_specs=[
            pl.BlockSpec((RB, DH), lambda i: (i, 0)),
            pl.BlockSpec((RB, DH), lambda i: (i, 0)),
        ],
        out_shape=[
            jax.ShapeDtypeStruct((NN, DH), jnp.float32),
            jax.ShapeDtypeStruct((NN, DH), jnp.float32),
        ],
    )(p01, x, weight)


NCH = EPS // CH
GSZ = 5
NGRP = NCH // GSZ
NZC = NN // CH


def _agg_body(src4_hbm, dst4_hbm, z0_hbm, z1_hbm, agg_hbm,
              slab_sh, idx_d2, idx_s2, rows_v, sem_g, sem_s, sem_i):
    c = lax.axis_index("c")
    s = lax.axis_index("s")
    zeros16 = jnp.zeros((16,), jnp.float32)

    def _zrow(r, carry):
        for k in range(DH // 16):
            rows_v[0, r, pl.ds(k * 16, 16)] = zeros16
        return carry
    lax.fori_loop(0, CH, _zrow, 0)
    for k in range((NZC + NS - 1) // NS):
        cid = k * NS + s

        @pl.when(cid < NZC)
        def _():
            pltpu.async_copy(rows_v.at[0], slab_sh.at[pl.ds(cid * CH, CH)],
                             sem_g)
    for k in range((NZC + NS - 1) // NS):
        cid = k * NS + s

        @pl.when(cid < NZC)
        def _():
            pltpu.make_async_copy(rows_v.at[0],
                                  slab_sh.at[pl.ds(cid * CH, CH)],
                                  sem_g).wait()

    plsc.subcore_barrier()

    def _run(z_ref):
        pltpu.sync_copy(dst4_hbm.at[s, 0], idx_d2.at[0])
        pltpu.sync_copy(src4_hbm.at[s, 0], idx_s2.at[0])

        def _group(h, carry):
            for p in range(2):
                gg = h * 2 + p

                for b in range(GSZ):
                    @pl.when(gg > 0)
                    def _():
                        pltpu.make_async_copy(
                            rows_v.at[b], slab_sh.at[idx_s2.at[p, b]],
                            sem_s).wait()
                    pltpu.async_copy(
                        z_ref.at[idx_d2.at[p, b]], rows_v.at[b], sem_g)

                @pl.when(gg + 1 < NGRP)
                def _():
                    pltpu.async_copy(dst4_hbm.at[s, gg + 1],
                                     idx_d2.at[1 - p], sem_i)
                    pltpu.async_copy(src4_hbm.at[s, gg + 1],
                                     idx_s2.at[1 - p], sem_i)

                for b in range(GSZ):
                    pltpu.make_async_copy(
                        z_ref.at[idx_d2.at[p, b]], rows_v.at[b], sem_g).wait()
                    pltpu.async_copy(
                        rows_v.at[b], slab_sh.at[idx_s2.at[p, b]],
                        sem_s, add=True)

                @pl.when(gg + 1 < NGRP)
                def _():
                    pltpu.make_async_copy(dst4_hbm.at[s, gg + 1],
                                          idx_d2.at[1 - p], sem_i).wait()
                    pltpu.make_async_copy(src4_hbm.at[s, gg + 1],
                                          idx_s2.at[1 - p], sem_i).wait()
            return carry
        lax.fori_loop(0, NGRP // 2, _group, 0)
        for b in range(GSZ):
            pltpu.make_async_copy(
                rows_v.at[b], slab_sh.at[idx_s2.at[1, b]], sem_s).wait()

    @pl.when(c == 0)
    def _():
        _run(z0_hbm)

    @pl.when(c == 1)
    def _():
        _run(z1_hbm)

    plsc.subcore_barrier()

    nk = (NZC + NS - 1) // NS
    for k in range(nk):
        cid = k * NS + s
        b = k % GSZ

        @pl.when(cid < NZC)
        def _():
            if k >= GSZ:
                pltpu.make_async_copy(
                    rows_v.at[b], agg_hbm.at[c, pl.ds(cid * CH, CH)],
                    sem_s).wait()
            pltpu.async_copy(slab_sh.at[pl.ds(cid * CH, CH)], rows_v.at[b],
                             sem_g).wait()
            pltpu.async_copy(rows_v.at[b], agg_hbm.at[c, pl.ds(cid * CH, CH)],
                             sem_s)
    for b in range(GSZ):
        pltpu.make_async_copy(rows_v.at[b],
                              agg_hbm.at[c, pl.ds(s * CH, CH)],
                              sem_s).wait()


_agg_kernel = pl.kernel(
    _agg_body,
    out_type=jax.ShapeDtypeStruct((NC, NN, DH), jnp.float32),
    mesh=_sc_mesh(),
    compiler_params=pltpu.CompilerParams(needs_layout_passes=False),
    scratch_types=[
        pltpu.VMEM_SHARED((NN, DH), jnp.float32),
        pltpu.VMEM((2, GSZ, CH), jnp.int32),
        pltpu.VMEM((2, GSZ, CH), jnp.int32),
        pltpu.VMEM((GSZ, CH, DH), jnp.float32),
        pltpu.SemaphoreType.DMA,
        pltpu.SemaphoreType.DMA,
        pltpu.SemaphoreType.DMA,
    ],
)


def _fin_body(p01_ref, bias_ref, agg_ref, out_ref):
    pd = p01_ref[...]
    deg = pd[:, :1] + pd[:, 1:]
    dis = jnp.where(deg > 0, lax.rsqrt(deg), 0.0)
    out_ref[:, :DH] = agg_ref[0] * dis + bias_ref[:, :DH]
    out_ref[:, DH:] = agg_ref[1] * dis + bias_ref[:, DH:]


def _fin_call(p01, bias2d, agg):
    return pl.pallas_call(
        _fin_body,
        grid=(NRB,),
        in_specs=[
            pl.BlockSpec((RB, 2), lambda i: (i, 0)),
            pl.BlockSpec((1, DOUT), lambda i: (0, 0)),
            pl.BlockSpec((NC, RB, DH), lambda i: (0, i, 0)),
        ],
        out_specs=pl.BlockSpec((RB, DOUT), lambda i: (i, 0)),
        out_shape=jax.ShapeDtypeStruct((NN, DOUT), jnp.float32),
    )(p01, bias2d, agg)


def kernel(x, edge_index, weight, bias):
    edge_index = edge_index.astype(jnp.int32)
    pdeg = _deg_kernel(edge_index)
    p01 = jnp.stack([pdeg[:NP], pdeg[NP:]], axis=1)
    z0, z1 = _mm_call(p01, x, weight)
    src4 = edge_index[0].reshape(NS, NGRP, GSZ, CH)
    dst4 = edge_index[1].reshape(NS, NGRP, GSZ, CH)
    agg = _agg_kernel(src4, dst4, z0, z1)
    return _fin_call(p01, bias.reshape(1, DOUT), agg)
